# baseline (speedup 1.0000x reference)
import functools

import jax
import jax.numpy as jnp
from jax import lax
from jax.experimental import pallas as pl
from jax.experimental.pallas import tpu as pltpu
from jax.experimental.pallas import tpu_sc as plsc

VOCAB = 5231
VPAD = 5232
EMB = 300
OUT_DIM = 37
BATCH = 16384
SEQ = 2
NSLAB = 3

_NC = 2
_NS = 16
_NW = _NC * _NS
_B_PER_W = BATCH // _NW

_VT = 1744


def _precompute_body(table_ref, wh_ref, bh_ref, *t_refs):
    t = table_ref[...]

    def slab(w_full, bias, lo, width):
        c = jnp.dot(
            t, w_full[:, lo : lo + width], preferred_element_type=jnp.float32
        )
        if bias is not None:
            c = c + bias[:, lo : lo + width]
        if width < 128:
            c = jnp.concatenate(
                [c, jnp.zeros((c.shape[0], 128 - width), jnp.float32)], axis=1
            )
        return c

    wa = wh_ref[0:EMB, :]
    wb = wh_ref[EMB:, :]
    for kk in range(NSLAB):
        lo = 128 * kk
        width = min(128, EMB - lo)
        t_refs[kk][...] = slab(wa, bh_ref, lo, width)
        t_refs[NSLAB + kk][...] = slab(wb, None, lo, width)


def _precompute(table, wh, bh):
    grid = (VPAD // _VT,)
    return pl.pallas_call(
        _precompute_body,
        grid=grid,
        in_specs=[
            pl.BlockSpec((_VT, EMB), lambda i: (i, 0)),
            pl.BlockSpec((2 * EMB, EMB), lambda i: (0, 0)),
            pl.BlockSpec((1, EMB), lambda i: (0, 0)),
        ],
        out_specs=[pl.BlockSpec((_VT, 128), lambda i: (i, 0))] * (2 * NSLAB),
        out_shape=[jax.ShapeDtypeStruct((VPAD, 128), jnp.float32)] * (2 * NSLAB),
    )(table, wh, bh)


_NSPLIT = 2
_BS = BATCH // _NSPLIT
_B_PER_W_S = _BS // _NW


def _sc_gather_body(
    t00, t01, t02, t10, t11, t12, idx0_hbm, idx1_hbm, out_hbm, i0_v, i1_v, rows_v, sem
):
    wid = lax.axis_index("s") * _NC + lax.axis_index("c")
    base = wid * _B_PER_W_S
    pltpu.sync_copy(idx0_hbm.at[pl.ds(base, _B_PER_W_S)], i0_v)
    pltpu.sync_copy(idx1_hbm.at[pl.ds(base, _B_PER_W_S)], i1_v)
    t0s = (t00, t01, t02)
    t1s = (t10, t11, t12)
    for k in range(NSLAB):
        pltpu.async_copy(t0s[k].at[i0_v], rows_v, sem).wait()
        pltpu.async_copy(t1s[k].at[i1_v], rows_v, sem, add=True).wait()
        pltpu.sync_copy(rows_v, out_hbm.at[pl.ds(k * _BS + base, _B_PER_W_S)])


_sc_gather = functools.partial(
    pl.kernel,
    out_type=jax.ShapeDtypeStruct((NSLAB * _BS, 128), jnp.float32),
    mesh=plsc.VectorSubcoreMesh(core_axis_name="c", subcore_axis_name="s"),
    scratch_types=[
        pltpu.VMEM((_B_PER_W_S,), jnp.int32),
        pltpu.VMEM((_B_PER_W_S,), jnp.int32),
        pltpu.VMEM((_B_PER_W_S, 128), jnp.float32),
        pltpu.SemaphoreType.DMA,
    ],
)(_sc_gather_body)


def _mlp_body(hp_ref, wo_ref, bo_ref, out_ref):
    s0 = jax.nn.sigmoid(hp_ref[0])
    s1 = jax.nn.sigmoid(hp_ref[1])
    s2 = jax.nn.sigmoid(hp_ref[2])[:, 0 : EMB - 256]
    wo = wo_ref[...]
    logits = (
        jnp.dot(s0, wo[0:128, :], preferred_element_type=jnp.float32)
        + jnp.dot(s1, wo[128:256, :], preferred_element_type=jnp.float32)
        + jnp.dot(s2, wo[256:EMB, :], preferred_element_type=jnp.float32)
        + bo_ref[...]
    )
    m = jnp.max(logits, axis=-1, keepdims=True)
    s = logits - m
    lse = jnp.log(jnp.sum(jnp.exp(s), axis=-1, keepdims=True))
    out_ref[...] = jnp.transpose(s - lse)


_TB = 2048


def _mlp(hp, wo, bo):
    grid = _BS // _TB
    return pl.pallas_call(
        _mlp_body,
        grid=(grid,),
        in_specs=[
            pl.BlockSpec((NSLAB, _TB, 128), lambda i: (0, i, 0)),
            pl.BlockSpec((EMB, OUT_DIM), lambda i: (0, 0)),
            pl.BlockSpec((1, OUT_DIM), lambda i: (0, 0)),
        ],
        out_specs=pl.BlockSpec((OUT_DIM, _TB), lambda i: (0, i)),
        out_shape=jax.ShapeDtypeStruct((OUT_DIM, _BS), jnp.float32),
    )(hp, wo, bo)


def kernel(x, table, W_h, b_h, W_o, b_o):
    xi = x.astype(jnp.int32)
    tabs = _precompute(table, W_h, b_h.reshape(1, EMB))
    bo = b_o.reshape(1, OUT_DIM)
    outs = []
    for sidx in range(_NSPLIT):
        sl = slice(sidx * _BS, (sidx + 1) * _BS)
        hp = _sc_gather(*tabs, xi[sl, 0], xi[sl, 1])
        hp3 = hp.reshape(NSLAB, _BS, 128)
        outs.append(_mlp(hp3, W_o, bo))
    return jnp.concatenate(outs, axis=1).T


# speedup vs baseline: 1.1082x; 1.1082x over previous
import functools

import jax
import jax.numpy as jnp
from jax import lax
from jax.experimental import pallas as pl
from jax.experimental.pallas import tpu as pltpu
from jax.experimental.pallas import tpu_sc as plsc

VOCAB = 5231
VPAD = 5232
EMB = 300
OUT_DIM = 37
BATCH = 16384
SEQ = 2
NSLAB = 3

_NC = 2
_NS = 16
_NW = _NC * _NS
_B_PER_W = BATCH // _NW

_VT = 1744


def _precompute_body(table_ref, wh_ref, bh_ref, *t_refs):
    t = table_ref[...]

    def slab(w_full, bias, lo, width):
        c = jnp.dot(
            t, w_full[:, lo : lo + width], preferred_element_type=jnp.float32
        )
        if bias is not None:
            c = c + bias[:, lo : lo + width]
        if width < 128:
            c = jnp.concatenate(
                [c, jnp.zeros((c.shape[0], 128 - width), jnp.float32)], axis=1
            )
        return c

    wa = wh_ref[0:EMB, :]
    wb = wh_ref[EMB:, :]
    for kk in range(NSLAB):
        lo = 128 * kk
        width = min(128, EMB - lo)
        t_refs[kk][...] = slab(wa, bh_ref, lo, width)
        t_refs[NSLAB + kk][...] = slab(wb, None, lo, width)


def _precompute(table, wh, bh):
    grid = (VPAD // _VT,)
    return pl.pallas_call(
        _precompute_body,
        grid=grid,
        in_specs=[
            pl.BlockSpec((_VT, EMB), lambda i: (i, 0)),
            pl.BlockSpec((2 * EMB, EMB), lambda i: (0, 0)),
            pl.BlockSpec((1, EMB), lambda i: (0, 0)),
        ],
        out_specs=[pl.BlockSpec((_VT, 128), lambda i: (i, 0))] * (2 * NSLAB),
        out_shape=[jax.ShapeDtypeStruct((VPAD, 128), jnp.float32)] * (2 * NSLAB),
    )(table, wh, bh)


_NSPLIT = 1
_BS = BATCH // _NSPLIT
_B_PER_W_S = _BS // _NW


def _sc_gather_body(
    t00, t01, t02, t10, t11, t12, idx0_hbm, idx1_hbm, out_hbm, i0_v, i1_v, rows_v, sem
):
    wid = lax.axis_index("s") * _NC + lax.axis_index("c")
    base = wid * _B_PER_W_S
    pltpu.sync_copy(idx0_hbm.at[pl.ds(base, _B_PER_W_S)], i0_v)
    pltpu.sync_copy(idx1_hbm.at[pl.ds(base, _B_PER_W_S)], i1_v)
    t0s = (t00, t01, t02)
    t1s = (t10, t11, t12)
    for k in range(NSLAB):
        pltpu.async_copy(t0s[k].at[i0_v], rows_v, sem).wait()
        pltpu.async_copy(t1s[k].at[i1_v], rows_v, sem, add=True).wait()
        pltpu.sync_copy(rows_v, out_hbm.at[pl.ds(k * _BS + base, _B_PER_W_S)])


_sc_gather = functools.partial(
    pl.kernel,
    out_type=jax.ShapeDtypeStruct((NSLAB * _BS, 128), jnp.float32),
    mesh=plsc.VectorSubcoreMesh(core_axis_name="c", subcore_axis_name="s"),
    scratch_types=[
        pltpu.VMEM((_B_PER_W_S,), jnp.int32),
        pltpu.VMEM((_B_PER_W_S,), jnp.int32),
        pltpu.VMEM((_B_PER_W_S, 128), jnp.float32),
        pltpu.SemaphoreType.DMA,
    ],
)(_sc_gather_body)


def _mlp_body(hp_ref, wo_ref, bo_ref, out_ref):
    s0 = jax.nn.sigmoid(hp_ref[0])
    s1 = jax.nn.sigmoid(hp_ref[1])
    s2 = jax.nn.sigmoid(hp_ref[2])[:, 0 : EMB - 256]
    wo = wo_ref[...]
    logits = (
        jnp.dot(s0, wo[0:128, :], preferred_element_type=jnp.float32)
        + jnp.dot(s1, wo[128:256, :], preferred_element_type=jnp.float32)
        + jnp.dot(s2, wo[256:EMB, :], preferred_element_type=jnp.float32)
        + bo_ref[...]
    )
    m = jnp.max(logits, axis=-1, keepdims=True)
    s = logits - m
    lse = jnp.log(jnp.sum(jnp.exp(s), axis=-1, keepdims=True))
    out_ref[...] = jnp.transpose(s - lse)


_TB = 2048


def _mlp(hp, wo, bo):
    grid = _BS // _TB
    return pl.pallas_call(
        _mlp_body,
        grid=(grid,),
        in_specs=[
            pl.BlockSpec((NSLAB, _TB, 128), lambda i: (0, i, 0)),
            pl.BlockSpec((EMB, OUT_DIM), lambda i: (0, 0)),
            pl.BlockSpec((1, OUT_DIM), lambda i: (0, 0)),
        ],
        out_specs=pl.BlockSpec((OUT_DIM, _TB), lambda i: (0, i)),
        out_shape=jax.ShapeDtypeStruct((OUT_DIM, _BS), jnp.float32),
    )(hp, wo, bo)


def kernel(x, table, W_h, b_h, W_o, b_o):
    xi = x.astype(jnp.int32)
    tabs = _precompute(table, W_h, b_h.reshape(1, EMB))
    bo = b_o.reshape(1, OUT_DIM)
    outs = []
    for sidx in range(_NSPLIT):
        sl = slice(sidx * _BS, (sidx + 1) * _BS)
        hp = _sc_gather(*tabs, xi[sl, 0], xi[sl, 1])
        hp3 = hp.reshape(NSLAB, _BS, 128)
        outs.append(_mlp(hp3, W_o, bo))
    return jnp.concatenate(outs, axis=1).T


# speedup vs baseline: 1.1203x; 1.0109x over previous
import functools

import jax
import jax.numpy as jnp
from jax import lax
from jax.experimental import pallas as pl
from jax.experimental.pallas import tpu as pltpu
from jax.experimental.pallas import tpu_sc as plsc

VOCAB = 5231
VPAD = 5232
EMB = 300
OUT_DIM = 37
BATCH = 16384
SEQ = 2
NSLAB = 3

_NC = 2
_NS = 16
_NW = _NC * _NS
_B_PER_W = BATCH // _NW

_VT = 1744


def _precompute_body(table_ref, wh_ref, bh_ref, *t_refs):
    t = table_ref[...]

    def slab(w_full, bias, lo, width):
        c = jnp.dot(
            t, w_full[:, lo : lo + width], preferred_element_type=jnp.float32
        )
        if bias is not None:
            c = c + bias[:, lo : lo + width]
        if width < 128:
            c = jnp.concatenate(
                [c, jnp.zeros((c.shape[0], 128 - width), jnp.float32)], axis=1
            )
        return c

    wa = wh_ref[0:EMB, :]
    wb = wh_ref[EMB:, :]
    for kk in range(NSLAB):
        lo = 128 * kk
        width = min(128, EMB - lo)
        t_refs[kk][...] = slab(wa, bh_ref, lo, width)
        t_refs[NSLAB + kk][...] = slab(wb, None, lo, width)


def _precompute(table, wh, bh):
    grid = (VPAD // _VT,)
    return pl.pallas_call(
        _precompute_body,
        grid=grid,
        in_specs=[
            pl.BlockSpec((_VT, EMB), lambda i: (i, 0)),
            pl.BlockSpec((2 * EMB, EMB), lambda i: (0, 0)),
            pl.BlockSpec((1, EMB), lambda i: (0, 0)),
        ],
        out_specs=[pl.BlockSpec((_VT, 128), lambda i: (i, 0))] * (2 * NSLAB),
        out_shape=[jax.ShapeDtypeStruct((VPAD, 128), jnp.float32)] * (2 * NSLAB),
    )(table, wh, bh)


_NSPLIT = 1
_BS = BATCH // _NSPLIT
_B_PER_W_S = _BS // _NW


def _sc_gather_body(
    t00, t01, t02, t10, t11, t12, idx0_hbm, idx1_hbm, out_hbm, i0_v, i1_v, rows_v, sem
):
    wid = lax.axis_index("s") * _NC + lax.axis_index("c")
    base = wid * _B_PER_W_S
    pltpu.sync_copy(idx0_hbm.at[pl.ds(base, _B_PER_W_S)], i0_v)
    pltpu.sync_copy(idx1_hbm.at[pl.ds(base, _B_PER_W_S)], i1_v)
    t0s = (t00, t01, t02)
    t1s = (t10, t11, t12)
    for k in range(NSLAB):
        pltpu.async_copy(t0s[k].at[i0_v], rows_v, sem).wait()
        pltpu.async_copy(t1s[k].at[i1_v], rows_v, sem, add=True).wait()
        pltpu.sync_copy(rows_v, out_hbm.at[pl.ds(k * _BS + base, _B_PER_W_S)])


_sc_gather = functools.partial(
    pl.kernel,
    out_type=jax.ShapeDtypeStruct((NSLAB * _BS, 128), jnp.float32),
    mesh=plsc.VectorSubcoreMesh(core_axis_name="c", subcore_axis_name="s"),
    scratch_types=[
        pltpu.VMEM((_B_PER_W_S,), jnp.int32),
        pltpu.VMEM((_B_PER_W_S,), jnp.int32),
        pltpu.VMEM((_B_PER_W_S, 128), jnp.float32),
        pltpu.SemaphoreType.DMA,
    ],
)(_sc_gather_body)


def _mlp_body(hp_ref, wo_ref, bo_ref, out_ref):
    s0 = jax.nn.sigmoid(hp_ref[0])
    s1 = jax.nn.sigmoid(hp_ref[1])
    s2 = jax.nn.sigmoid(hp_ref[2])[:, 0 : EMB - 256]
    wo = wo_ref[...]
    logits = (
        jnp.dot(s0, wo[0:128, :], preferred_element_type=jnp.float32)
        + jnp.dot(s1, wo[128:256, :], preferred_element_type=jnp.float32)
        + jnp.dot(s2, wo[256:EMB, :], preferred_element_type=jnp.float32)
        + bo_ref[...]
    )
    m = jnp.max(logits, axis=-1, keepdims=True)
    s = logits - m
    lse = jnp.log(jnp.sum(jnp.exp(s), axis=-1, keepdims=True))
    out_ref[...] = jnp.transpose(s - lse)


_TB = 4096


def _mlp(hp, wo, bo):
    grid = _BS // _TB
    return pl.pallas_call(
        _mlp_body,
        grid=(grid,),
        in_specs=[
            pl.BlockSpec((NSLAB, _TB, 128), lambda i: (0, i, 0)),
            pl.BlockSpec((EMB, OUT_DIM), lambda i: (0, 0)),
            pl.BlockSpec((1, OUT_DIM), lambda i: (0, 0)),
        ],
        out_specs=pl.BlockSpec((OUT_DIM, _TB), lambda i: (0, i)),
        out_shape=jax.ShapeDtypeStruct((OUT_DIM, _BS), jnp.float32),
    )(hp, wo, bo)


def kernel(x, table, W_h, b_h, W_o, b_o):
    xi = x.astype(jnp.int32)
    tabs = _precompute(table, W_h, b_h.reshape(1, EMB))
    bo = b_o.reshape(1, OUT_DIM)
    outs = []
    for sidx in range(_NSPLIT):
        sl = slice(sidx * _BS, (sidx + 1) * _BS)
        hp = _sc_gather(*tabs, xi[sl, 0], xi[sl, 1])
        hp3 = hp.reshape(NSLAB, _BS, 128)
        outs.append(_mlp(hp3, W_o, bo))
    return jnp.concatenate(outs, axis=1).T
